# baseline (speedup 1.0000x reference)
import functools

import jax
import jax.numpy as jnp
from jax import lax
from jax.experimental import pallas as pl
from jax.experimental.pallas import tpu as pltpu
from jax.experimental.pallas import tpu_sc as plsc

VOCAB = 1000000
HIDDEN = 64
MAX_LEN = 512
BATCH = 4096
SEQ = 200

NC = 2
NS = 16
NW = NC * NS

N_ROWS = BATCH * SEQ
ROWS_PER_W = N_ROWS // NW
CHUNK = 1024
N_CHUNKS = ROWS_PER_W // CHUNK


def _make_gather():
    mesh = plsc.VectorSubcoreMesh(core_axis_name="c", subcore_axis_name="s",
                                  num_cores=NC, num_subcores=NS)

    @functools.partial(
        pl.kernel,
        out_type=jax.ShapeDtypeStruct((N_ROWS, HIDDEN), jnp.float32),
        mesh=mesh,
        scratch_types=[
            pltpu.VMEM((CHUNK,), jnp.int32),
            pltpu.VMEM((CHUNK, HIDDEN), jnp.float32),
            pltpu.SemaphoreType.DMA,
        ],
        compiler_params=pltpu.CompilerParams(use_tc_tiling_on_sc=False),
    )
    def gather_k(table_hbm, idx_hbm, out_hbm, idx_v, rows_v, sem):
        wid = lax.axis_index("s") * NC + lax.axis_index("c")

        def body(i, carry):
            base = wid * ROWS_PER_W + i * CHUNK
            pltpu.sync_copy(idx_hbm.at[pl.ds(base, CHUNK)], idx_v)
            pltpu.async_copy(table_hbm.at[idx_v], rows_v, sem).wait()
            pltpu.sync_copy(rows_v, out_hbm.at[pl.ds(base, CHUNK)])
            return carry

        lax.fori_loop(0, N_CHUNKS, body, 0)

    return gather_k


_gather = _make_gather()

_BB = 32


def _ln_block(emb_ref, pos_ref, gamma_ref, beta_ref, out_ref):
    x = emb_ref[...] + pos_ref[...][None, :, :]
    mean = jnp.mean(x, axis=-1, keepdims=True)
    xc = x - mean
    var = jnp.mean(xc * xc, axis=-1, keepdims=True)
    inv = lax.rsqrt(var + 1e-5)
    out_ref[...] = xc * inv * gamma_ref[...] + beta_ref[...]


def _ln(emb, pos, gamma, beta):
    grid = (BATCH // _BB,)
    return pl.pallas_call(
        _ln_block,
        grid=grid,
        in_specs=[
            pl.BlockSpec((_BB, SEQ, HIDDEN), lambda i: (i, 0, 0)),
            pl.BlockSpec((SEQ, HIDDEN), lambda i: (0, 0)),
            pl.BlockSpec((HIDDEN,), lambda i: (0,)),
            pl.BlockSpec((HIDDEN,), lambda i: (0,)),
        ],
        out_specs=pl.BlockSpec((_BB, SEQ, HIDDEN), lambda i: (i, 0, 0)),
        out_shape=jax.ShapeDtypeStruct((BATCH, SEQ, HIDDEN), jnp.float32),
    )(emb, pos, gamma, beta)


def kernel(input_ids, word_table, pos_table, gamma, beta):
    idx = input_ids.reshape(-1).astype(jnp.int32)
    emb = _gather(word_table, idx)
    emb = emb.reshape(BATCH, SEQ, HIDDEN)
    pos = pos_table[:SEQ]
    return _ln(emb, pos, gamma, beta)


# speedup vs baseline: 1.1790x; 1.1790x over previous
import functools

import jax
import jax.numpy as jnp
from jax import lax
from jax.experimental import pallas as pl
from jax.experimental.pallas import tpu as pltpu
from jax.experimental.pallas import tpu_sc as plsc

VOCAB = 1000000
HIDDEN = 64
MAX_LEN = 512
BATCH = 4096
SEQ = 200

NC = 2
NS = 16
NW = NC * NS

N_ROWS = BATCH * SEQ
N_PACKED = N_ROWS // 2
PK_PER_W = N_PACKED // NW
CP = 400
N_CHUNKS = PK_PER_W // CP
N_PAIRS = N_CHUNKS // 2


def _make_gather():
    mesh = plsc.VectorSubcoreMesh(core_axis_name="c", subcore_axis_name="s",
                                  num_cores=NC, num_subcores=NS)

    @functools.partial(
        pl.kernel,
        out_type=jax.ShapeDtypeStruct((N_PACKED, 2 * HIDDEN), jnp.float32),
        mesh=mesh,
        scratch_types=[
            pltpu.VMEM((CP,), jnp.int32),
            pltpu.VMEM((CP,), jnp.int32),
            pltpu.VMEM((CP,), jnp.int32),
            pltpu.VMEM((CP,), jnp.int32),
            pltpu.VMEM((CP, HIDDEN), jnp.float32),
            pltpu.VMEM((CP, HIDDEN), jnp.float32),
            pltpu.VMEM((CP, HIDDEN), jnp.float32),
            pltpu.VMEM((CP, HIDDEN), jnp.float32),
            pltpu.SemaphoreType.DMA,
            pltpu.SemaphoreType.DMA,
        ],
        compiler_params=pltpu.CompilerParams(use_tc_tiling_on_sc=False),
    )
    def gather_k(table_hbm, idx_hbm, out_hbm,
                 idxl0, idxr0, idxl1, idxr1,
                 rowsl0, rowsr0, rowsl1, rowsr1, sem0, sem1):
        wid = lax.axis_index("s") * NC + lax.axis_index("c")
        w_base = wid * PK_PER_W

        def fire(pbase, idxl, idxr, rowsl, rowsr, sem):
            pltpu.sync_copy(idx_hbm.at[0, pl.ds(pbase, CP)], idxl)
            pltpu.sync_copy(idx_hbm.at[1, pl.ds(pbase, CP)], idxr)
            pltpu.async_copy(table_hbm.at[idxl], rowsl, sem)
            pltpu.async_copy(table_hbm.at[idxr], rowsr, sem)

        def drain(pbase, idxl, idxr, rowsl, rowsr, sem):
            pltpu.make_async_copy(table_hbm.at[idxl], rowsl, sem).wait()
            pltpu.make_async_copy(table_hbm.at[idxr], rowsr, sem).wait()
            pltpu.sync_copy(rowsl,
                            out_hbm.at[pl.ds(pbase, CP), pl.ds(0, HIDDEN)])
            pltpu.sync_copy(rowsr,
                            out_hbm.at[pl.ds(pbase, CP), pl.ds(HIDDEN, HIDDEN)])

        fire(w_base, idxl0, idxr0, rowsl0, rowsr0, sem0)

        def body(j, carry):
            pbase0 = w_base + 2 * j * CP
            pbase1 = pbase0 + CP
            fire(pbase1, idxl1, idxr1, rowsl1, rowsr1, sem1)
            drain(pbase0, idxl0, idxr0, rowsl0, rowsr0, sem0)

            @pl.when(j + 1 < N_PAIRS)
            def _():
                fire(pbase1 + CP, idxl0, idxr0, rowsl0, rowsr0, sem0)

            drain(pbase1, idxl1, idxr1, rowsl1, rowsr1, sem1)
            return carry

        lax.fori_loop(0, N_PAIRS, body, 0)

    return gather_k


_gather = _make_gather()

_BB = 32
_HB = BATCH // 2
_NK = _HB // _BB
_R = _BB * SEQ


def _ln_block(emb_ref, pos_ref, gamma_ref, beta_ref, out_ref):
    h = pl.program_id(1)
    e = emb_ref[...].reshape(_BB, SEQ, 2 * HIDDEN) + pos_ref[...][None, :, :]
    x = jnp.where(h == 0, e[:, :, :HIDDEN], e[:, :, HIDDEN:])
    mean = jnp.mean(x, axis=-1, keepdims=True)
    xc = x - mean
    var = jnp.mean(xc * xc, axis=-1, keepdims=True)
    inv = lax.rsqrt(var + 1e-5)
    out_ref[...] = xc * inv * gamma_ref[...] + beta_ref[...]


def _ln(emb2, pos2, gamma, beta):
    return pl.pallas_call(
        _ln_block,
        grid=(_NK, 2),
        in_specs=[
            pl.BlockSpec((_R, 2 * HIDDEN), lambda k, h: (k, 0)),
            pl.BlockSpec((SEQ, 2 * HIDDEN), lambda k, h: (0, 0)),
            pl.BlockSpec((HIDDEN,), lambda k, h: (0,)),
            pl.BlockSpec((HIDDEN,), lambda k, h: (0,)),
        ],
        out_specs=pl.BlockSpec((_BB, SEQ, HIDDEN),
                               lambda k, h: (h * _NK + k, 0, 0)),
        out_shape=jax.ShapeDtypeStruct((BATCH, SEQ, HIDDEN), jnp.float32),
    )(emb2, pos2, gamma, beta)


def kernel(input_ids, word_table, pos_table, gamma, beta):
    ids = input_ids.astype(jnp.int32)
    idx2 = ids.reshape(2, N_PACKED)
    emb2 = _gather(word_table, idx2)
    pos = pos_table[:SEQ]
    pos2 = jnp.concatenate([pos, pos], axis=1)
    return _ln(emb2, pos2, gamma, beta)
